# Initial kernel scaffold; baseline (speedup 1.0000x reference)
import jax
import jax.numpy as jnp
from jax.experimental import pallas as pl


def kernel(x, edge_index, edge_attr, node_attr, batch, additional_message_features, additional_node_features, W_m1, b_m1, W_m2, b_m2, W_u1, b_u1, W_u2, b_u2):
    raise NotImplementedError("write your pallas kernel here")


# speedup vs baseline: 3.2842x; 3.2842x over previous
import functools

import jax
import jax.numpy as jnp
from jax import lax
from jax.experimental import pallas as pl
from jax.experimental.pallas import tpu as pltpu
from jax.experimental.pallas import tpu_sc as plsc

_NC = 2
_NS = 16
_CH = 128


def _proj_body(x_ref, wd_ref, ws_ref, pd_ref, ps_ref):
    xb = x_ref[...]
    pd_ref[...] = jnp.dot(xb, wd_ref[...], preferred_element_type=jnp.float32)
    ps_ref[...] = jnp.dot(xb, ws_ref[...], preferred_element_type=jnp.float32)


def _proj(N, D, BN):
    return pl.pallas_call(
        _proj_body,
        grid=(N // BN,),
        in_specs=[
            pl.BlockSpec((BN, D), lambda i: (i, 0)),
            pl.BlockSpec((D, D), lambda i: (0, 0)),
            pl.BlockSpec((D, D), lambda i: (0, 0)),
        ],
        out_specs=[
            pl.BlockSpec((BN, D), lambda i: (i, 0)),
            pl.BlockSpec((BN, D), lambda i: (i, 0)),
        ],
        out_shape=[
            jax.ShapeDtypeStruct((N, D), jnp.float32),
            jax.ShapeDtypeStruct((N, D), jnp.float32),
        ],
    )


def _sc_gather(N, E, D):
    nw = _NC * _NS
    ew = E // nw
    assert E % nw == 0 and ew % 8 == 0
    nch = -(-ew // _CH)

    mesh = plsc.VectorSubcoreMesh(core_axis_name="c", subcore_axis_name="s")

    @functools.partial(
        pl.kernel,
        mesh=mesh,
        out_type=[
            jax.ShapeDtypeStruct((E, D), jnp.float32),
            jax.ShapeDtypeStruct((E, D), jnp.float32),
        ],
        scratch_types=[
            pltpu.VMEM((_CH,), jnp.int32),
            pltpu.VMEM((_CH,), jnp.int32),
            pltpu.VMEM((_CH, D), jnp.float32),
            pltpu.VMEM((_CH, D), jnp.float32),
            pltpu.SemaphoreType.DMA,
            pltpu.SemaphoreType.DMA,
        ],
    )
    def gather_k(pd_hbm, ps_hbm, dst_hbm, src_hbm, gd_hbm, gs_hbm,
                 dstv, srcv, rdv, rsv, sem_d, sem_s):
        c = lax.axis_index("c")
        s = lax.axis_index("s")
        wid = s * _NC + c
        base_w = wid * ew

        @pl.loop(0, nch)
        def _chunk(i):
            off = base_w + jnp.minimum(i * _CH, ew - _CH)
            pltpu.sync_copy(dst_hbm.at[pl.ds(off, _CH)], dstv)
            pltpu.sync_copy(src_hbm.at[pl.ds(off, _CH)], srcv)
            cpd = pltpu.async_copy(pd_hbm.at[dstv], rdv, sem_d)
            cps = pltpu.async_copy(ps_hbm.at[srcv], rsv, sem_s)
            cpd.wait()
            cps.wait()
            pltpu.sync_copy(rdv, gd_hbm.at[pl.ds(off, _CH)])
            pltpu.sync_copy(rsv, gs_hbm.at[pl.ds(off, _CH)])

    return gather_k


def _edge_body(gd_ref, gs_ref, amf_ref, ea_ref, w1t_ref, b1_ref,
               w2_ref, b2_ref, m_ref):
    ea = ea_ref[...]
    z1 = (gd_ref[...] + gs_ref[...]
          + jnp.dot(jnp.concatenate([amf_ref[...], ea], axis=-1), w1t_ref[...],
                    preferred_element_type=jnp.float32)
          + b1_ref[...])
    h = jax.nn.silu(z1)
    z2 = jnp.dot(jnp.concatenate([h, ea], axis=-1), w2_ref[...],
                 preferred_element_type=jnp.float32) + b2_ref[...]
    m_ref[...] = jax.nn.silu(z2)


def _edge_mlp(E, D, DA, BE):
    return pl.pallas_call(
        _edge_body,
        grid=(E // BE,),
        in_specs=[
            pl.BlockSpec((BE, D), lambda i: (i, 0)),
            pl.BlockSpec((BE, D), lambda i: (i, 0)),
            pl.BlockSpec((BE, DA), lambda i: (i, 0)),
            pl.BlockSpec((BE, DA), lambda i: (i, 0)),
            pl.BlockSpec((2 * DA, D), lambda i: (0, 0)),
            pl.BlockSpec((1, D), lambda i: (0, 0)),
            pl.BlockSpec((D + DA, D), lambda i: (0, 0)),
            pl.BlockSpec((1, D), lambda i: (0, 0)),
        ],
        out_specs=pl.BlockSpec((BE, D), lambda i: (i, 0)),
        out_shape=jax.ShapeDtypeStruct((E, D), jnp.float32),
    )


def _sc_scatter(N, E, D):
    nw = _NC * _NS
    ew = E // nw
    assert E % nw == 0 and ew % 8 == 0
    nfull = ew // _CH
    tail = ew - nfull * _CH
    npad = -(-N // (8 * _NS)) * (8 * _NS)
    rps = npad // _NS

    mesh = plsc.VectorSubcoreMesh(core_axis_name="c", subcore_axis_name="s")

    scratch = [
        pltpu.VMEM((_CH,), jnp.int32),
        pltpu.VMEM((_CH, D), jnp.float32),
        pltpu.VMEM_SHARED((npad, D), jnp.float32),
    ]
    if tail:
        scratch += [
            pltpu.VMEM((tail,), jnp.int32),
            pltpu.VMEM((tail, D), jnp.float32),
        ]

    @functools.partial(
        pl.kernel,
        mesh=mesh,
        out_type=jax.ShapeDtypeStruct((_NC * npad, D), jnp.float32),
        scratch_types=scratch,
    )
    def scatter_k(m_hbm, dst_hbm, zeros_hbm, out_hbm, dstv, mv, acc, *tl):
        c = lax.axis_index("c")
        s = lax.axis_index("s")
        wid = s * _NC + c
        base_w = wid * ew

        pltpu.sync_copy(zeros_hbm.at[pl.ds(s * rps, rps)],
                        acc.at[pl.ds(s * rps, rps)])
        plsc.subcore_barrier()

        @pl.loop(0, nfull)
        def _chunk(i):
            off = base_w + i * _CH
            pltpu.sync_copy(dst_hbm.at[pl.ds(off, _CH)], dstv)
            pltpu.sync_copy(m_hbm.at[pl.ds(off, _CH)], mv)
            pltpu.sync_copy(mv, acc.at[dstv], add=True)

        if tail:
            dstv_t, mv_t = tl
            off = base_w + nfull * _CH
            pltpu.sync_copy(dst_hbm.at[pl.ds(off, tail)], dstv_t)
            pltpu.sync_copy(m_hbm.at[pl.ds(off, tail)], mv_t)
            pltpu.sync_copy(mv_t, acc.at[dstv_t], add=True)

        plsc.subcore_barrier()
        pltpu.sync_copy(acc.at[pl.ds(s * rps, rps)],
                        out_hbm.at[pl.ds(c * npad + s * rps, rps)])

    return scatter_k


def _update_body(x_ref, p_ref, anf_ref, na_ref, wu1_ref, bu1_ref,
                 wu2_ref, bu2_ref, o_ref):
    na = na_ref[...]
    agg = p_ref[0] + p_ref[1]
    u_in = jnp.concatenate([x_ref[...], agg, anf_ref[...], na], axis=-1)
    u = jax.nn.silu(jnp.dot(u_in, wu1_ref[...],
                            preferred_element_type=jnp.float32) + bu1_ref[...])
    o_ref[...] = jnp.dot(jnp.concatenate([u, na], axis=-1), wu2_ref[...],
                         preferred_element_type=jnp.float32) + bu2_ref[...]


def _update(N, D, DA, BN):
    return pl.pallas_call(
        _update_body,
        grid=(N // BN,),
        in_specs=[
            pl.BlockSpec((BN, D), lambda i: (i, 0)),
            pl.BlockSpec((_NC, BN, D), lambda i: (0, i, 0)),
            pl.BlockSpec((BN, DA), lambda i: (i, 0)),
            pl.BlockSpec((BN, DA), lambda i: (i, 0)),
            pl.BlockSpec((2 * D + 2 * DA, D), lambda i: (0, 0)),
            pl.BlockSpec((1, D), lambda i: (0, 0)),
            pl.BlockSpec((D + DA, D), lambda i: (0, 0)),
            pl.BlockSpec((1, D), lambda i: (0, 0)),
        ],
        out_specs=pl.BlockSpec((BN, D), lambda i: (i, 0)),
        out_shape=jax.ShapeDtypeStruct((N, D), jnp.float32),
    )


def kernel(x, edge_index, edge_attr, node_attr, batch,
           additional_message_features, additional_node_features,
           W_m1, b_m1, W_m2, b_m2, W_u1, b_u1, W_u2, b_u2):
    del batch
    N, D = x.shape
    E, DA = edge_attr.shape
    src = edge_index[0]
    dst = edge_index[1]

    pd, ps = _proj(N, D, 2000)(x, W_m1[:D], W_m1[D:2 * D])
    gd, gs = _sc_gather(N, E, D)(pd, ps, dst, src)
    m = _edge_mlp(E, D, DA, 2000)(
        gd, gs, additional_message_features, edge_attr,
        W_m1[2 * D:], b_m1.reshape(1, D), W_m2, b_m2.reshape(1, D))
    npad = -(-N // (8 * _NS)) * (8 * _NS)
    parts = _sc_scatter(N, E, D)(m, dst, jnp.zeros((npad, D), jnp.float32))
    p = parts.reshape(_NC, npad, D)[:, :N]
    out = _update(N, D, DA, 2000)(
        x, p, additional_node_features, node_attr,
        W_u1, b_u1.reshape(1, D), W_u2, b_u2.reshape(1, D))
    return out


# speedup vs baseline: 3.8129x; 1.1610x over previous
import functools

import jax
import jax.numpy as jnp
from jax import lax
from jax.experimental import pallas as pl
from jax.experimental.pallas import tpu as pltpu
from jax.experimental.pallas import tpu_sc as plsc

_NC = 2
_NS = 16
_CH = 128


def _proj_body(x_ref, wd_ref, ws_ref, pd_ref, ps_ref):
    xb = x_ref[...]
    pd_ref[...] = jnp.dot(xb, wd_ref[...], preferred_element_type=jnp.float32)
    ps_ref[...] = jnp.dot(xb, ws_ref[...], preferred_element_type=jnp.float32)


def _proj(N, D, BN):
    return pl.pallas_call(
        _proj_body,
        grid=(N // BN,),
        in_specs=[
            pl.BlockSpec((BN, D), lambda i: (i, 0)),
            pl.BlockSpec((D, D), lambda i: (0, 0)),
            pl.BlockSpec((D, D), lambda i: (0, 0)),
        ],
        out_specs=[
            pl.BlockSpec((BN, D), lambda i: (i, 0)),
            pl.BlockSpec((BN, D), lambda i: (i, 0)),
        ],
        out_shape=[
            jax.ShapeDtypeStruct((N, D), jnp.float32),
            jax.ShapeDtypeStruct((N, D), jnp.float32),
        ],
    )


def _sc_gather(N, E, D):
    nw = _NC * _NS
    ew = E // nw
    assert E % nw == 0 and ew % 8 == 0
    nch = -(-ew // _CH)

    mesh = plsc.VectorSubcoreMesh(core_axis_name="c", subcore_axis_name="s")

    @functools.partial(
        pl.kernel,
        mesh=mesh,
        out_type=[
            jax.ShapeDtypeStruct((E, D), jnp.float32),
            jax.ShapeDtypeStruct((E, D), jnp.float32),
        ],
        scratch_types=[
            pltpu.VMEM((_CH,), jnp.int32), pltpu.VMEM((_CH,), jnp.int32),
            pltpu.VMEM((_CH,), jnp.int32), pltpu.VMEM((_CH,), jnp.int32),
            pltpu.VMEM((_CH, D), jnp.float32), pltpu.VMEM((_CH, D), jnp.float32),
            pltpu.VMEM((_CH, D), jnp.float32), pltpu.VMEM((_CH, D), jnp.float32),
            pltpu.SemaphoreType.DMA, pltpu.SemaphoreType.DMA,
            pltpu.SemaphoreType.DMA, pltpu.SemaphoreType.DMA,
        ],
    )
    def gather_k(pd_hbm, ps_hbm, dst_hbm, src_hbm, gd_hbm, gs_hbm,
                 dstv0, srcv0, dstv1, srcv1, rdv0, rsv0, rdv1, rsv1,
                 gsem0, gsem1, wsem0, wsem1):
        c = lax.axis_index("c")
        s = lax.axis_index("s")
        wid = s * _NC + c
        base_w = wid * ew
        bufs = ((dstv0, srcv0, rdv0, rsv0, gsem0, wsem0),
                (dstv1, srcv1, rdv1, rsv1, gsem1, wsem1))

        def off(ch):
            return base_w + jnp.minimum(ch * _CH, ew - _CH)

        def fire(ch, b):
            dstv, srcv, rdv, rsv, gsem, _ = bufs[b]
            o = off(ch)
            pltpu.sync_copy(dst_hbm.at[pl.ds(o, _CH)], dstv)
            pltpu.sync_copy(src_hbm.at[pl.ds(o, _CH)], srcv)
            pltpu.async_copy(pd_hbm.at[dstv], rdv, gsem)
            pltpu.async_copy(ps_hbm.at[srcv], rsv, gsem)

        def drain_and_write(ch, b):
            dstv, srcv, rdv, rsv, gsem, wsem = bufs[b]
            o = off(ch)
            pltpu.make_async_copy(pd_hbm.at[dstv], rdv, gsem).wait()
            pltpu.make_async_copy(ps_hbm.at[srcv], rsv, gsem).wait()
            pltpu.async_copy(rdv, gd_hbm.at[pl.ds(o, _CH)], wsem)
            pltpu.async_copy(rsv, gs_hbm.at[pl.ds(o, _CH)], wsem)

        def wait_writes(ch, b):
            _, _, rdv, rsv, _, wsem = bufs[b]
            o = off(ch)
            pltpu.make_async_copy(rdv, gd_hbm.at[pl.ds(o, _CH)], wsem).wait()
            pltpu.make_async_copy(rsv, gs_hbm.at[pl.ds(o, _CH)], wsem).wait()

        fire(0, 0)

        @pl.loop(0, 2 * ((nch + 1) // 2), step=2)
        def _pair(i):
            for b in (0, 1):
                ch = i + b

                @pl.when(ch + 1 < nch)
                def _():
                    @pl.when(ch + 1 >= 2)
                    def _():
                        wait_writes(ch - 1, 1 - b)
                    fire(ch + 1, 1 - b)

                @pl.when(ch < nch)
                def _():
                    drain_and_write(ch, b)

        if nch >= 2:
            wait_writes(nch - 2, (nch - 2) % 2)
        wait_writes(nch - 1, (nch - 1) % 2)

    return gather_k


def _edge_body(gd_ref, gs_ref, amf_ref, ea_ref, w1t_ref, b1_ref,
               w2_ref, b2_ref, m_ref):
    ea = ea_ref[...]
    z1 = (gd_ref[...] + gs_ref[...]
          + jnp.dot(jnp.concatenate([amf_ref[...], ea], axis=-1), w1t_ref[...],
                    preferred_element_type=jnp.float32)
          + b1_ref[...])
    h = jax.nn.silu(z1)
    z2 = jnp.dot(jnp.concatenate([h, ea], axis=-1), w2_ref[...],
                 preferred_element_type=jnp.float32) + b2_ref[...]
    m_ref[...] = jax.nn.silu(z2)


def _edge_mlp(E, D, DA, BE):
    return pl.pallas_call(
        _edge_body,
        grid=(E // BE,),
        in_specs=[
            pl.BlockSpec((BE, D), lambda i: (i, 0)),
            pl.BlockSpec((BE, D), lambda i: (i, 0)),
            pl.BlockSpec((BE, DA), lambda i: (i, 0)),
            pl.BlockSpec((BE, DA), lambda i: (i, 0)),
            pl.BlockSpec((2 * DA, D), lambda i: (0, 0)),
            pl.BlockSpec((1, D), lambda i: (0, 0)),
            pl.BlockSpec((D + DA, D), lambda i: (0, 0)),
            pl.BlockSpec((1, D), lambda i: (0, 0)),
        ],
        out_specs=pl.BlockSpec((BE, D), lambda i: (i, 0)),
        out_shape=jax.ShapeDtypeStruct((E, D), jnp.float32),
    )


def _sc_scatter(N, E, D):
    nw = _NC * _NS
    ew = E // nw
    assert E % nw == 0 and ew % 8 == 0
    nfull = ew // _CH
    tail = ew - nfull * _CH
    npad = -(-N // (8 * _NS)) * (8 * _NS)
    rps = npad // _NS

    mesh = plsc.VectorSubcoreMesh(core_axis_name="c", subcore_axis_name="s")

    scratch = [
        pltpu.VMEM((_CH,), jnp.int32), pltpu.VMEM((_CH,), jnp.int32),
        pltpu.VMEM((_CH, D), jnp.float32), pltpu.VMEM((_CH, D), jnp.float32),
        pltpu.VMEM_SHARED((npad, D), jnp.float32),
        pltpu.SemaphoreType.DMA, pltpu.SemaphoreType.DMA,
        pltpu.SemaphoreType.DMA, pltpu.SemaphoreType.DMA,
    ]
    if tail:
        scratch += [
            pltpu.VMEM((tail,), jnp.int32),
            pltpu.VMEM((tail, D), jnp.float32),
        ]

    @functools.partial(
        pl.kernel,
        mesh=mesh,
        out_type=jax.ShapeDtypeStruct((_NC * npad, D), jnp.float32),
        scratch_types=scratch,
    )
    def scatter_k(m_hbm, dst_hbm, zeros_hbm, out_hbm,
                  dstv0, dstv1, mv0, mv1, acc,
                  lsem0, lsem1, ssem0, ssem1, *tl):
        c = lax.axis_index("c")
        s = lax.axis_index("s")
        wid = s * _NC + c
        base_w = wid * ew
        bufs = ((dstv0, mv0, lsem0, ssem0), (dstv1, mv1, lsem1, ssem1))

        pltpu.sync_copy(zeros_hbm.at[pl.ds(s * rps, rps)],
                        acc.at[pl.ds(s * rps, rps)])
        plsc.subcore_barrier()

        def fire_loads(ch, b):
            dstv, mv, lsem, _ = bufs[b]
            o = base_w + ch * _CH
            pltpu.async_copy(dst_hbm.at[pl.ds(o, _CH)], dstv, lsem)
            pltpu.async_copy(m_hbm.at[pl.ds(o, _CH)], mv, lsem)

        def fire_scatter(ch, b):
            dstv, mv, lsem, ssem = bufs[b]
            o = base_w + ch * _CH
            pltpu.make_async_copy(dst_hbm.at[pl.ds(o, _CH)], dstv, lsem).wait()
            pltpu.make_async_copy(m_hbm.at[pl.ds(o, _CH)], mv, lsem).wait()
            pltpu.async_copy(mv, acc.at[dstv], ssem, add=True)

        def wait_scatter(b):
            dstv, mv, _, ssem = bufs[b]
            pltpu.make_async_copy(mv, acc.at[dstv], ssem).wait()

        if nfull:
            fire_loads(0, 0)

            @pl.loop(0, 2 * ((nfull + 1) // 2), step=2)
            def _pair(i):
                for b in (0, 1):
                    ch = i + b

                    @pl.when(ch + 1 < nfull)
                    def _():
                        @pl.when(ch + 1 >= 2)
                        def _():
                            wait_scatter(1 - b)
                        fire_loads(ch + 1, 1 - b)

                    @pl.when(ch < nfull)
                    def _():
                        fire_scatter(ch, b)

            if nfull >= 2:
                wait_scatter((nfull - 2) % 2)
            wait_scatter((nfull - 1) % 2)

        if tail:
            dstv_t, mv_t = tl
            o = base_w + nfull * _CH
            pltpu.sync_copy(dst_hbm.at[pl.ds(o, tail)], dstv_t)
            pltpu.sync_copy(m_hbm.at[pl.ds(o, tail)], mv_t)
            pltpu.sync_copy(mv_t, acc.at[dstv_t], add=True)

        plsc.subcore_barrier()
        pltpu.sync_copy(acc.at[pl.ds(s * rps, rps)],
                        out_hbm.at[pl.ds(c * npad + s * rps, rps)])

    return scatter_k


def _update_body(x_ref, p_ref, anf_ref, na_ref, wu1_ref, bu1_ref,
                 wu2_ref, bu2_ref, o_ref):
    na = na_ref[...]
    agg = p_ref[0] + p_ref[1]
    u_in = jnp.concatenate([x_ref[...], agg, anf_ref[...], na], axis=-1)
    u = jax.nn.silu(jnp.dot(u_in, wu1_ref[...],
                            preferred_element_type=jnp.float32) + bu1_ref[...])
    o_ref[...] = jnp.dot(jnp.concatenate([u, na], axis=-1), wu2_ref[...],
                         preferred_element_type=jnp.float32) + bu2_ref[...]


def _update(N, D, DA, BN):
    return pl.pallas_call(
        _update_body,
        grid=(N // BN,),
        in_specs=[
            pl.BlockSpec((BN, D), lambda i: (i, 0)),
            pl.BlockSpec((_NC, BN, D), lambda i: (0, i, 0)),
            pl.BlockSpec((BN, DA), lambda i: (i, 0)),
            pl.BlockSpec((BN, DA), lambda i: (i, 0)),
            pl.BlockSpec((2 * D + 2 * DA, D), lambda i: (0, 0)),
            pl.BlockSpec((1, D), lambda i: (0, 0)),
            pl.BlockSpec((D + DA, D), lambda i: (0, 0)),
            pl.BlockSpec((1, D), lambda i: (0, 0)),
        ],
        out_specs=pl.BlockSpec((BN, D), lambda i: (i, 0)),
        out_shape=jax.ShapeDtypeStruct((N, D), jnp.float32),
    )


def kernel(x, edge_index, edge_attr, node_attr, batch,
           additional_message_features, additional_node_features,
           W_m1, b_m1, W_m2, b_m2, W_u1, b_u1, W_u2, b_u2):
    del batch
    N, D = x.shape
    E, DA = edge_attr.shape
    src = edge_index[0]
    dst = edge_index[1]

    pd, ps = _proj(N, D, 2000)(x, W_m1[:D], W_m1[D:2 * D])
    gd, gs = _sc_gather(N, E, D)(pd, ps, dst, src)
    m = _edge_mlp(E, D, DA, 2000)(
        gd, gs, additional_message_features, edge_attr,
        W_m1[2 * D:], b_m1.reshape(1, D), W_m2, b_m2.reshape(1, D))
    npad = -(-N // (8 * _NS)) * (8 * _NS)
    parts = _sc_scatter(N, E, D)(m, dst, jnp.zeros((npad, D), jnp.float32))
    p = parts.reshape(_NC, npad, D)[:, :N]
    out = _update(N, D, DA, 2000)(
        x, p, additional_node_features, node_attr,
        W_u1, b_u1.reshape(1, D), W_u2, b_u2.reshape(1, D))
    return out


# speedup vs baseline: 3.8174x; 1.0012x over previous
import functools

import jax
import jax.numpy as jnp
from jax import lax
from jax.experimental import pallas as pl
from jax.experimental.pallas import tpu as pltpu
from jax.experimental.pallas import tpu_sc as plsc

_NC = 2
_NS = 16
_CH = 128
_NB = 3
_SNB = 2


def _proj_body(x_ref, wd_ref, ws_ref, pd_ref, ps_ref):
    xb = x_ref[...]
    pd_ref[...] = jnp.dot(xb, wd_ref[...], preferred_element_type=jnp.float32)
    ps_ref[...] = jnp.dot(xb, ws_ref[...], preferred_element_type=jnp.float32)


def _proj(N, D, BN):
    return pl.pallas_call(
        _proj_body,
        grid=(N // BN,),
        in_specs=[
            pl.BlockSpec((BN, D), lambda i: (i, 0)),
            pl.BlockSpec((D, D), lambda i: (0, 0)),
            pl.BlockSpec((D, D), lambda i: (0, 0)),
        ],
        out_specs=[
            pl.BlockSpec((BN, D), lambda i: (i, 0)),
            pl.BlockSpec((BN, D), lambda i: (i, 0)),
        ],
        out_shape=[
            jax.ShapeDtypeStruct((N, D), jnp.float32),
            jax.ShapeDtypeStruct((N, D), jnp.float32),
        ],
    )


def _sc_gather(N, E, D):
    nw = _NC * _NS
    ew = E // nw
    assert E % nw == 0 and ew % 8 == 0
    nch = -(-ew // _CH)

    mesh = plsc.VectorSubcoreMesh(core_axis_name="c", subcore_axis_name="s")

    @functools.partial(
        pl.kernel,
        mesh=mesh,
        out_type=[
            jax.ShapeDtypeStruct((E, D), jnp.float32),
            jax.ShapeDtypeStruct((E, D), jnp.float32),
        ],
        scratch_types=(
            [pltpu.VMEM((_CH,), jnp.int32) for _ in range(2 * _NB)]
            + [pltpu.VMEM((_CH, D), jnp.float32) for _ in range(2 * _NB)]
            + [pltpu.SemaphoreType.DMA for _ in range(2 * _NB)]
        ),
    )
    def gather_k(pd_hbm, ps_hbm, dst_hbm, src_hbm, gd_hbm, gs_hbm, *scr):
        c = lax.axis_index("c")
        s = lax.axis_index("s")
        wid = s * _NC + c
        base_w = wid * ew
        idxs = scr[:2 * _NB]
        rows = scr[2 * _NB:4 * _NB]
        sems = scr[4 * _NB:]
        bufs = tuple(
            (idxs[2 * b], idxs[2 * b + 1], rows[2 * b], rows[2 * b + 1],
             sems[2 * b], sems[2 * b + 1])
            for b in range(_NB))
        la = _NB - 1

        def off(ch):
            return base_w + jnp.minimum(ch * _CH, ew - _CH)

        def fire(ch, b):
            dstv, srcv, rdv, rsv, gsem, _ = bufs[b]
            o = off(ch)
            pltpu.sync_copy(dst_hbm.at[pl.ds(o, _CH)], dstv)
            pltpu.sync_copy(src_hbm.at[pl.ds(o, _CH)], srcv)
            pltpu.async_copy(pd_hbm.at[dstv], rdv, gsem)
            pltpu.async_copy(ps_hbm.at[srcv], rsv, gsem)

        def drain_and_write(ch, b):
            dstv, srcv, rdv, rsv, gsem, wsem = bufs[b]
            o = off(ch)
            pltpu.make_async_copy(pd_hbm.at[dstv], rdv, gsem).wait()
            pltpu.make_async_copy(ps_hbm.at[srcv], rsv, gsem).wait()
            pltpu.async_copy(rdv, gd_hbm.at[pl.ds(o, _CH)], wsem)
            pltpu.async_copy(rsv, gs_hbm.at[pl.ds(o, _CH)], wsem)

        def wait_writes(ch, b):
            _, _, rdv, rsv, _, wsem = bufs[b]
            o = off(ch)
            pltpu.make_async_copy(rdv, gd_hbm.at[pl.ds(o, _CH)], wsem).wait()
            pltpu.make_async_copy(rsv, gs_hbm.at[pl.ds(o, _CH)], wsem).wait()

        for p in range(min(la, nch)):
            fire(p, p)

        @pl.loop(0, _NB * (-(-nch // _NB)), step=_NB)
        def _blk(i):
            for b in range(_NB):
                ch = i + b
                nxt = ch + la
                fb = (b + la) % _NB

                @pl.when(nxt < nch)
                def _():
                    @pl.when(nxt >= _NB)
                    def _():
                        wait_writes(nxt - _NB, fb)
                    fire(nxt, fb)

                @pl.when(ch < nch)
                def _():
                    drain_and_write(ch, b)

        for q in range(max(0, nch - _NB), nch):
            wait_writes(q, q % _NB)

    return gather_k


def _edge_body(gd_ref, gs_ref, amf_ref, ea_ref, w1t_ref, b1_ref,
               w2_ref, b2_ref, m_ref):
    ea = ea_ref[...]
    z1 = (gd_ref[...] + gs_ref[...]
          + jnp.dot(jnp.concatenate([amf_ref[...], ea], axis=-1), w1t_ref[...],
                    preferred_element_type=jnp.float32)
          + b1_ref[...])
    h = jax.nn.silu(z1)
    z2 = jnp.dot(jnp.concatenate([h, ea], axis=-1), w2_ref[...],
                 preferred_element_type=jnp.float32) + b2_ref[...]
    m_ref[...] = jax.nn.silu(z2)


def _edge_mlp(E, D, DA, BE):
    return pl.pallas_call(
        _edge_body,
        grid=(E // BE,),
        in_specs=[
            pl.BlockSpec((BE, D), lambda i: (i, 0)),
            pl.BlockSpec((BE, D), lambda i: (i, 0)),
            pl.BlockSpec((BE, DA), lambda i: (i, 0)),
            pl.BlockSpec((BE, DA), lambda i: (i, 0)),
            pl.BlockSpec((2 * DA, D), lambda i: (0, 0)),
            pl.BlockSpec((1, D), lambda i: (0, 0)),
            pl.BlockSpec((D + DA, D), lambda i: (0, 0)),
            pl.BlockSpec((1, D), lambda i: (0, 0)),
        ],
        out_specs=pl.BlockSpec((BE, D), lambda i: (i, 0)),
        out_shape=jax.ShapeDtypeStruct((E, D), jnp.float32),
    )


def _sc_scatter(N, E, D):
    nw = _NC * _NS
    ew = E // nw
    assert E % nw == 0 and ew % 8 == 0
    nfull = ew // _CH
    tail = ew - nfull * _CH
    npad = -(-N // (8 * _NS)) * (8 * _NS)
    rps = npad // _NS

    mesh = plsc.VectorSubcoreMesh(core_axis_name="c", subcore_axis_name="s")

    scratch = (
        [pltpu.VMEM((_CH,), jnp.int32) for _ in range(_SNB)]
        + [pltpu.VMEM((_CH, D), jnp.float32) for _ in range(_SNB)]
        + [pltpu.SemaphoreType.DMA for _ in range(2 * _SNB)]
        + [pltpu.VMEM_SHARED((npad, D), jnp.float32)]
    )
    if tail:
        scratch += [
            pltpu.VMEM((tail,), jnp.int32),
            pltpu.VMEM((tail, D), jnp.float32),
        ]

    @functools.partial(
        pl.kernel,
        mesh=mesh,
        out_type=jax.ShapeDtypeStruct((_NC * npad, D), jnp.float32),
        scratch_types=scratch,
    )
    def scatter_k(m_hbm, dst_hbm, zeros_hbm, out_hbm, *scr):
        c = lax.axis_index("c")
        s = lax.axis_index("s")
        wid = s * _NC + c
        base_w = wid * ew
        idxs = scr[:_SNB]
        mrows = scr[_SNB:2 * _SNB]
        sems = scr[2 * _SNB:4 * _SNB]
        acc = scr[4 * _SNB]
        tl = scr[4 * _SNB + 1:]
        bufs = tuple((idxs[b], mrows[b], sems[2 * b], sems[2 * b + 1])
                     for b in range(_SNB))
        la = _SNB - 1

        pltpu.sync_copy(zeros_hbm.at[pl.ds(s * rps, rps)],
                        acc.at[pl.ds(s * rps, rps)])
        plsc.subcore_barrier()

        def fire_loads(ch, b):
            dstv, mv, lsem, _ = bufs[b]
            o = base_w + ch * _CH
            pltpu.async_copy(dst_hbm.at[pl.ds(o, _CH)], dstv, lsem)
            pltpu.async_copy(m_hbm.at[pl.ds(o, _CH)], mv, lsem)

        def fire_scatter(ch, b):
            dstv, mv, lsem, ssem = bufs[b]
            o = base_w + ch * _CH
            pltpu.make_async_copy(dst_hbm.at[pl.ds(o, _CH)], dstv, lsem).wait()
            pltpu.make_async_copy(m_hbm.at[pl.ds(o, _CH)], mv, lsem).wait()
            pltpu.async_copy(mv, acc.at[dstv], ssem, add=True)

        def wait_scatter(b):
            dstv, mv, _, ssem = bufs[b]
            pltpu.make_async_copy(mv, acc.at[dstv], ssem).wait()

        if nfull:
            for p in range(min(la, nfull)):
                fire_loads(p, p)

            @pl.loop(0, _SNB * (-(-nfull // _SNB)), step=_SNB)
            def _blk(i):
                for b in range(_SNB):
                    ch = i + b
                    nxt = ch + la
                    fb = (b + la) % _SNB

                    @pl.when(nxt < nfull)
                    def _():
                        @pl.when(nxt >= _SNB)
                        def _():
                            wait_scatter(fb)
                        fire_loads(nxt, fb)

                    @pl.when(ch < nfull)
                    def _():
                        fire_scatter(ch, b)

            for q in range(max(0, nfull - _SNB), nfull):
                wait_scatter(q % _SNB)

        if tail:
            dstv_t, mv_t = tl
            o = base_w + nfull * _CH
            pltpu.sync_copy(dst_hbm.at[pl.ds(o, tail)], dstv_t)
            pltpu.sync_copy(m_hbm.at[pl.ds(o, tail)], mv_t)
            pltpu.sync_copy(mv_t, acc.at[dstv_t], add=True)

        plsc.subcore_barrier()
        pltpu.sync_copy(acc.at[pl.ds(s * rps, rps)],
                        out_hbm.at[pl.ds(c * npad + s * rps, rps)])

    return scatter_k


def _update_body(x_ref, p_ref, anf_ref, na_ref, wu1_ref, bu1_ref,
                 wu2_ref, bu2_ref, o_ref):
    na = na_ref[...]
    agg = p_ref[0] + p_ref[1]
    u_in = jnp.concatenate([x_ref[...], agg, anf_ref[...], na], axis=-1)
    u = jax.nn.silu(jnp.dot(u_in, wu1_ref[...],
                            preferred_element_type=jnp.float32) + bu1_ref[...])
    o_ref[...] = jnp.dot(jnp.concatenate([u, na], axis=-1), wu2_ref[...],
                         preferred_element_type=jnp.float32) + bu2_ref[...]


def _update(N, D, DA, BN):
    return pl.pallas_call(
        _update_body,
        grid=(N // BN,),
        in_specs=[
            pl.BlockSpec((BN, D), lambda i: (i, 0)),
            pl.BlockSpec((_NC, BN, D), lambda i: (0, i, 0)),
            pl.BlockSpec((BN, DA), lambda i: (i, 0)),
            pl.BlockSpec((BN, DA), lambda i: (i, 0)),
            pl.BlockSpec((2 * D + 2 * DA, D), lambda i: (0, 0)),
            pl.BlockSpec((1, D), lambda i: (0, 0)),
            pl.BlockSpec((D + DA, D), lambda i: (0, 0)),
            pl.BlockSpec((1, D), lambda i: (0, 0)),
        ],
        out_specs=pl.BlockSpec((BN, D), lambda i: (i, 0)),
        out_shape=jax.ShapeDtypeStruct((N, D), jnp.float32),
    )


def kernel(x, edge_index, edge_attr, node_attr, batch,
           additional_message_features, additional_node_features,
           W_m1, b_m1, W_m2, b_m2, W_u1, b_u1, W_u2, b_u2):
    del batch
    N, D = x.shape
    E, DA = edge_attr.shape
    src = edge_index[0]
    dst = edge_index[1]

    pd, ps = _proj(N, D, 2000)(x, W_m1[:D], W_m1[D:2 * D])
    gd, gs = _sc_gather(N, E, D)(pd, ps, dst, src)
    m = _edge_mlp(E, D, DA, 2000)(
        gd, gs, additional_message_features, edge_attr,
        W_m1[2 * D:], b_m1.reshape(1, D), W_m2, b_m2.reshape(1, D))
    npad = -(-N // (8 * _NS)) * (8 * _NS)
    parts = _sc_scatter(N, E, D)(m, dst, jnp.zeros((npad, D), jnp.float32))
    p = parts.reshape(_NC, npad, D)[:, :N]
    out = _update(N, D, DA, 2000)(
        x, p, additional_node_features, node_attr,
        W_u1, b_u1.reshape(1, D), W_u2, b_u2.reshape(1, D))
    return out
